# baseline (device time: 29636 ns/iter reference)
import jax
import jax.numpy as jnp
from jax import lax
from jax.experimental import pallas as pl
from jax.experimental.pallas import tpu as pltpu

N_DEV = 4
B_LOC = 2
SQ = 256
SKV = 256
HQ = 16
H_BLK = 4
DH = 64
D_MODEL = 512
D_BLK = H_BLK * DH


def kernel(x, Wq, K_ext, V_ext, Wo):
    i = lax.axis_index("i")
    Ks = lax.dynamic_slice_in_dim(K_ext, B_LOC * i, B_LOC, axis=0)
    Vs = lax.dynamic_slice_in_dim(V_ext, B_LOC * i, B_LOC, axis=0)
    Kp = Ks.astype(jnp.bfloat16).transpose(0, 2, 1, 3).reshape(B_LOC * HQ, SKV, DH)
    Vp = Vs.astype(jnp.bfloat16).transpose(0, 2, 1, 3).reshape(B_LOC * HQ, SKV, DH)
    x_bf = x.astype(jnp.bfloat16)
    wq_bf = Wq.astype(jnp.bfloat16)
    wo_bf = Wo.astype(jnp.bfloat16)

    def body(x_ref, wq_ref, k_ref, v_ref, wo_ref, out_ref,
             wq_comm, wo_comm, send_sems, recv_sems, acc, ctx_buf):
        my = lax.axis_index("i")

        barrier = pltpu.get_barrier_semaphore()
        for k in range(1, N_DEV):
            pl.semaphore_signal(
                barrier, inc=1,
                device_id=(lax.rem(my + k, N_DEV),),
                device_id_type=pl.DeviceIdType.MESH,
            )
        pl.semaphore_wait(barrier, N_DEV - 1)

        sends = []
        for k in range(1, N_DEV):
            dst = lax.rem(my + k, N_DEV)
            r_wq = pltpu.make_async_remote_copy(
                src_ref=wq_ref,
                dst_ref=wq_comm.at[k - 1],
                send_sem=send_sems.at[2 * (k - 1)],
                recv_sem=recv_sems.at[2 * (k - 1)],
                device_id=(dst,),
                device_id_type=pl.DeviceIdType.MESH,
            )
            r_wo = pltpu.make_async_remote_copy(
                src_ref=wo_ref,
                dst_ref=wo_comm.at[k - 1],
                send_sem=send_sems.at[2 * (k - 1) + 1],
                recv_sem=recv_sems.at[2 * (k - 1) + 1],
                device_id=(dst,),
                device_id_type=pl.DeviceIdType.MESH,
            )
            r_wq.start()
            r_wo.start()
            sends.append(r_wq)
            sends.append(r_wo)

        qi = lax.broadcasted_iota(jnp.int32, (SQ, SKV), 0)
        ki = lax.broadcasted_iota(jnp.int32, (SQ, SKV), 1)
        mask = (jnp.abs(qi - ki) <= 128) | (ki < 32) | (qi < 32)

        x2d = x_ref[:].reshape(B_LOC * SQ, D_MODEL)

        def do_block(origin, wq_blk, wo_blk):
            q = lax.dot_general(
                x2d, wq_blk, (((1,), (0,)), ((), ())),
                preferred_element_type=jnp.float32,
            )
            q_bf = q.astype(jnp.bfloat16)
            for b in range(B_LOC):
                for h in range(H_BLK):
                    qh = q_bf[b * SQ:(b + 1) * SQ, h * DH:(h + 1) * DH]
                    idx = b * HQ + H_BLK * origin + h
                    kh = k_ref[idx]
                    vh = v_ref[idx]
                    s = lax.dot_general(
                        qh, kh, (((1,), (1,)), ((), ())),
                        preferred_element_type=jnp.float32,
                    ) * 0.125
                    s = jnp.where(mask, s, -1e9)
                    m = jnp.max(s, axis=1, keepdims=True)
                    w = jnp.exp(s - m)
                    w = w / jnp.sum(w, axis=1, keepdims=True)
                    ctx = lax.dot_general(
                        w.astype(jnp.bfloat16), vh, (((1,), (0,)), ((), ())),
                        preferred_element_type=jnp.float32,
                    )
                    ctx_buf[b * SQ:(b + 1) * SQ, h * DH:(h + 1) * DH] = (
                        ctx.astype(jnp.bfloat16)
                    )
            return lax.dot_general(
                ctx_buf[:], wo_blk, (((1,), (0,)), ((), ())),
                preferred_element_type=jnp.float32,
            )

        acc[:] = do_block(my, wq_ref[:], wo_ref[:])

        for k in (1, 3, 2):
            recv_wq = pltpu.make_async_remote_copy(
                src_ref=wq_ref,
                dst_ref=wq_comm.at[k - 1],
                send_sem=send_sems.at[2 * (k - 1)],
                recv_sem=recv_sems.at[2 * (k - 1)],
                device_id=(my,),
                device_id_type=pl.DeviceIdType.MESH,
            )
            recv_wo = pltpu.make_async_remote_copy(
                src_ref=wo_ref,
                dst_ref=wo_comm.at[k - 1],
                send_sem=send_sems.at[2 * (k - 1) + 1],
                recv_sem=recv_sems.at[2 * (k - 1) + 1],
                device_id=(my,),
                device_id_type=pl.DeviceIdType.MESH,
            )
            recv_wq.wait_recv()
            recv_wo.wait_recv()
            origin = lax.rem(my + (N_DEV - k), N_DEV)
            acc[:] += do_block(origin, wq_comm[k - 1], wo_comm[k - 1])

        for r in sends:
            r.wait_send()

        out_ref[:] = acc[:].reshape(B_LOC, SQ, D_MODEL)

    return pl.pallas_call(
        body,
        out_shape=jax.ShapeDtypeStruct((B_LOC, SQ, D_MODEL), jnp.float32),
        in_specs=[pl.BlockSpec(memory_space=pltpu.VMEM)] * 5,
        out_specs=pl.BlockSpec(memory_space=pltpu.VMEM),
        scratch_shapes=[
            pltpu.VMEM((N_DEV - 1, D_MODEL, D_BLK), jnp.bfloat16),
            pltpu.VMEM((N_DEV - 1, D_BLK, D_MODEL), jnp.bfloat16),
            pltpu.SemaphoreType.DMA((2 * (N_DEV - 1),)),
            pltpu.SemaphoreType.DMA((2 * (N_DEV - 1),)),
            pltpu.VMEM((B_LOC * SQ, D_MODEL), jnp.float32),
            pltpu.VMEM((B_LOC * SQ, D_BLK), jnp.bfloat16),
        ],
        compiler_params=pltpu.CompilerParams(collective_id=0),
    )(x_bf, wq_bf, Kp, Vp, wo_bf)
